# baseline (speedup 1.0000x reference)
import jax
import jax.numpy as jnp
from jax import lax
from jax.experimental import pallas as pl
from jax.experimental.pallas import tpu as pltpu
from jax.experimental.pallas import tpu_sc as plsc

BATCH = 16384
OUT_DIM = 128
EMBED_DIM = 64
CAT_DIM = OUT_DIM + EMBED_DIM
N_ROWS = 1000001
NC, NS, L = 2, 16, 16
NW = NC * NS
ROWS_PER_W = BATCH // NW
CHUNK = 128
GROUPS = CHUNK // L

SLAB = 256
SLABS_PER_W = 122
W_RANGE = SLABS_PER_W * SLAB
EXTRA_SLAB0 = NW * SLABS_PER_W * SLAB
EXTRA_SLAB1 = EXTRA_SLAB0 + SLAB
TAIL0 = EXTRA_SLAB1 + SLAB
TAIL_W = N_ROWS - TAIL0
G_ROWS = BATCH + NW

_RSQRT_MAGIC = 0x5F3759DF


def _inv_norm(s):
    s_safe = jnp.maximum(s, jnp.float32(1.2e-38))
    y = plsc.bitcast(
        jnp.int32(_RSQRT_MAGIC) - (plsc.bitcast(s_safe, jnp.int32) >> 1),
        jnp.float32)
    for _ in range(3):
        y = y * (jnp.float32(1.5) - jnp.float32(0.5) * s_safe * y * y)
    n = s * y
    return jnp.float32(1.0) / jnp.maximum(n, jnp.float32(1e-12))


def _scan_body(ann_hbm, emb_t_hbm, g_hbm,
               abuf, whits_r, whits_k, shits_r, shits_k,
               slab_a, slab_b, tail_v, r2_v, sem, sem_a, sem_b):
    wid = lax.axis_index("s") * NC + lax.axis_index("c")
    lo = wid * W_RANGE
    hi = jnp.where(wid == NW - 1, jnp.int32(N_ROWS), lo + W_RANGE)
    lanes = lax.iota(jnp.int32, L)
    dummy = (BATCH + wid) * EMBED_DIM

    def _append(cnt, dst_r, dst_k, rv, kv, m):
        plsc.store_compressed(dst_r.at[pl.ds(cnt, L)], rv, mask=m)
        plsc.store_compressed(dst_k.at[pl.ds(cnt, L)], kv, mask=m)
        npop = plsc.all_reduce_population_count(m)
        return cnt + npop[0]

    def scan_chunk(c8, cnt):
        pltpu.sync_copy(ann_hbm.at[pl.ds(c8 * 2048, 2048)], abuf)

        def scan_group(g, cnt):
            rv = abuf[pl.ds(g * L, L)]
            kv = c8 * 2048 + g * L + lanes
            m = (rv >= lo) & (rv < hi)
            return _append(cnt, whits_r, whits_k, rv, kv, m)

        return lax.fori_loop(0, 2048 // L, scan_group, cnt)

    cnt = lax.fori_loop(0, BATCH // 2048, scan_chunk, jnp.int32(0))

    def process_hits(col0, width, buf):
        def filt(h, scnt):
            pos = h * L + lanes
            rv = whits_r[pl.ds(h * L, L)]
            kv = whits_k[pl.ds(h * L, L)]
            m = (pos < cnt) & (rv >= col0) & (rv < col0 + width)
            return _append(scnt, shits_r, shits_k, rv, kv, m)

        scnt = lax.fori_loop(0, (cnt + L - 1) // L, filt, jnp.int32(0))
        nq = (scnt + L - 1) // L

        def extract(q, _):
            pos = q * L + lanes
            vm = pos < scnt
            rv = shits_r[pl.ds(q * L, L)]
            kv = shits_k[pl.ds(q * L, L)]
            rl = jnp.where(vm, rv - col0, 0)
            rowq = (q & 3) * L
            for d in range(EMBED_DIM):
                c = (d + lanes) & (EMBED_DIM - 1)
                vals = plsc.load_gather(buf, [c, rl], mask=vm)
                plsc.store_scatter(r2_v, [(rowq + lanes) * EMBED_DIM + c],
                                   vals, mask=vm)
            ksafe = jnp.where(vm, kv * EMBED_DIM, dummy)
            for j in range(L):
                koff = pl.multiple_of(ksafe[j], EMBED_DIM)
                pltpu.async_copy(
                    r2_v.at[pl.ds((rowq + j) * EMBED_DIM, EMBED_DIM)],
                    g_hbm.at[pl.ds(koff, EMBED_DIM)], sem)
            pltpu.make_async_copy(
                g_hbm.at[pl.ds(0, L * EMBED_DIM)],
                r2_v.at[pl.ds(0, L * EMBED_DIM)], sem).wait()
            return 0

        lax.fori_loop(0, nq, extract, 0)

    def prefetch(s, buf, sem_):
        pltpu.async_copy(emb_t_hbm.at[:, pl.ds(lo + s * SLAB, SLAB)],
                         buf, sem_)

    def wait_slab(buf, sem_):
        pltpu.make_async_copy(emb_t_hbm.at[:, pl.ds(0, SLAB)],
                              buf, sem_).wait()

    prefetch(jnp.int32(0), slab_a, sem_a)

    def main_slab(s, _):
        for parity, buf, smm, nbuf, nsem in (
                (0, slab_a, sem_a, slab_b, sem_b),
                (1, slab_b, sem_b, slab_a, sem_a)):
            @pl.when((s & 1) == parity)
            def _():
                wait_slab(buf, smm)

                @pl.when(s + 1 < SLABS_PER_W)
                def _():
                    prefetch(s + 1, nbuf, nsem)

                process_hits(lo + s * SLAB, SLAB, buf)

        return 0

    lax.fori_loop(0, SLABS_PER_W, main_slab, 0)

    @pl.when(wid == NW - 1)
    def _():
        pltpu.sync_copy(emb_t_hbm.at[:, pl.ds(EXTRA_SLAB0, SLAB)], slab_a)
        process_hits(jnp.int32(EXTRA_SLAB0), SLAB, slab_a)
        pltpu.sync_copy(emb_t_hbm.at[:, pl.ds(EXTRA_SLAB1, SLAB)], slab_b)
        process_hits(jnp.int32(EXTRA_SLAB1), SLAB, slab_b)
        pltpu.sync_copy(emb_t_hbm.at[:, pl.ds(TAIL0, TAIL_W)], tail_v)
        process_hits(jnp.int32(TAIL0), TAIL_W, tail_v)


def _norm_body(outs_t_hbm, g_hbm, out_t_hbm, obuf_t, gbuf, catbuf_t):
    wid = lax.axis_index("s") * NC + lax.axis_index("c")
    lanes = lax.iota(jnp.int32, L)

    for ch in range(ROWS_PER_W // CHUNK):
        base = wid * ROWS_PER_W + ch * CHUNK
        pltpu.sync_copy(outs_t_hbm.at[:, pl.ds(base, CHUNK)], obuf_t)
        pltpu.sync_copy(g_hbm.at[pl.ds(base * EMBED_DIM, CHUNK * EMBED_DIM)],
                        gbuf)

        def group(g, _):
            sl = pl.ds(g * L, L)
            accs = [None] * 8
            for col in range(OUT_DIM):
                v = obuf_t[col, sl]
                a = col & 7
                accs[a] = v * v if accs[a] is None else accs[a] + v * v
            acc = ((accs[0] + accs[1]) + (accs[2] + accs[3])) + (
                (accs[4] + accs[5]) + (accs[6] + accs[7]))
            io = _inv_norm(acc)
            rowbase = (g * L + lanes) * EMBED_DIM
            acc2s = [None] * 4
            for d in range(EMBED_DIM):
                c = (d + lanes) & (EMBED_DIM - 1)
                vals = plsc.load_gather(gbuf, [rowbase + c])
                a = d & 3
                acc2s[a] = (vals * vals if acc2s[a] is None
                            else acc2s[a] + vals * vals)
            acc2 = (acc2s[0] + acc2s[1]) + (acc2s[2] + acc2s[3])
            ie = _inv_norm(acc2)
            for col in range(OUT_DIM):
                catbuf_t[col, sl] = obuf_t[col, sl] * io
            for d in range(EMBED_DIM):
                c = (d + lanes) & (EMBED_DIM - 1)
                vals = plsc.load_gather(gbuf, [rowbase + c])
                plsc.store_scatter(catbuf_t, [OUT_DIM + c, g * L + lanes],
                                   vals * ie)
            return 0

        lax.fori_loop(0, GROUPS, group, 0)
        pltpu.sync_copy(catbuf_t, out_t_hbm.at[:, pl.ds(base, CHUNK)])


@jax.jit
def _crowd_concat(outputs, annotators, embedding):
    emb_t = embedding.T
    outs_t = outputs.T
    mesh = plsc.VectorSubcoreMesh(core_axis_name="c", subcore_axis_name="s")
    params = pltpu.CompilerParams(
        needs_layout_passes=False, use_tc_tiling_on_sc=True)

    g1 = pl.kernel(
        _scan_body,
        out_type=jax.ShapeDtypeStruct((G_ROWS * EMBED_DIM,), jnp.float32),
        mesh=mesh,
        scratch_types=[
            pltpu.VMEM((2048,), jnp.int32),
            pltpu.VMEM((BATCH,), jnp.int32),
            pltpu.VMEM((BATCH,), jnp.int32),
            pltpu.VMEM((BATCH,), jnp.int32),
            pltpu.VMEM((BATCH,), jnp.int32),
            pltpu.VMEM((EMBED_DIM, SLAB), jnp.float32),
            pltpu.VMEM((EMBED_DIM, SLAB), jnp.float32),
            pltpu.VMEM((EMBED_DIM, TAIL_W), jnp.float32),
            pltpu.VMEM((4 * L * EMBED_DIM,), jnp.float32),
            pltpu.SemaphoreType.DMA,
            pltpu.SemaphoreType.DMA,
            pltpu.SemaphoreType.DMA,
        ],
        compiler_params=params,
    )(annotators, emb_t)

    out_t = pl.kernel(
        _norm_body,
        out_type=jax.ShapeDtypeStruct((CAT_DIM, BATCH), jnp.float32),
        mesh=mesh,
        scratch_types=[
            pltpu.VMEM((OUT_DIM, CHUNK), jnp.float32),
            pltpu.VMEM((CHUNK * EMBED_DIM,), jnp.float32),
            pltpu.VMEM((CAT_DIM, CHUNK), jnp.float32),
        ],
        compiler_params=params,
    )(outs_t, g1)

    return out_t.T


def kernel(outputs, annotators, embedding):
    return _crowd_concat(outputs, annotators, embedding)


# speedup vs baseline: 1.2481x; 1.2481x over previous
import jax
import jax.numpy as jnp
from jax import lax
from jax.experimental import pallas as pl
from jax.experimental.pallas import tpu as pltpu
from jax.experimental.pallas import tpu_sc as plsc

BATCH = 16384
OUT_DIM = 128
EMBED_DIM = 64
CAT_DIM = OUT_DIM + EMBED_DIM
N_ROWS = 1000001
NC, NS, L = 2, 16, 16
NW = NC * NS
ROWS_PER_W = BATCH // NW
CHUNK = 256
GROUPS = CHUNK // L

SLAB = 512
SLABS_PER_W = 61
W_RANGE = SLABS_PER_W * SLAB
EXTRA_SLAB0 = NW * SLABS_PER_W * SLAB
TAIL0 = EXTRA_SLAB0 + SLAB
TAIL_W = N_ROWS - TAIL0
G_ROWS = BATCH + NW
RING = 16

_RSQRT_MAGIC = 0x5F3759DF


def _inv_norm(s):
    s_safe = jnp.maximum(s, jnp.float32(1.2e-38))
    y = plsc.bitcast(
        jnp.int32(_RSQRT_MAGIC) - (plsc.bitcast(s_safe, jnp.int32) >> 1),
        jnp.float32)
    for _ in range(3):
        y = y * (jnp.float32(1.5) - jnp.float32(0.5) * s_safe * y * y)
    n = s * y
    return jnp.float32(1.0) / jnp.maximum(n, jnp.float32(1e-12))


def _scan_body(ann_hbm, emb_t_hbm, g_hbm,
               abuf, whits_r, whits_k, shits_r, shits_k,
               slab_v, tail_v, r2_v, sem):
    wid = lax.axis_index("s") * NC + lax.axis_index("c")
    lo = wid * W_RANGE
    hi = jnp.where(wid == NW - 1, jnp.int32(N_ROWS), lo + W_RANGE)
    lanes = lax.iota(jnp.int32, L)
    dummy = (BATCH + wid) * EMBED_DIM

    def _append(cnt, dst_r, dst_k, rv, kv, m):
        plsc.store_compressed(dst_r.at[pl.ds(cnt, L)], rv, mask=m)
        plsc.store_compressed(dst_k.at[pl.ds(cnt, L)], kv, mask=m)
        npop = plsc.all_reduce_population_count(m)
        return cnt + npop[0]

    def scan_chunk(c8, cnt):
        pltpu.sync_copy(ann_hbm.at[pl.ds(c8 * 2048, 2048)], abuf)

        def scan_group(g, cnt):
            rv = abuf[pl.ds(g * L, L)]
            kv = c8 * 2048 + g * L + lanes
            m = (rv >= lo) & (rv < hi)
            return _append(cnt, whits_r, whits_k, rv, kv, m)

        return lax.fori_loop(0, 2048 // L, scan_group, cnt)

    cnt = lax.fori_loop(0, BATCH // 2048, scan_chunk, jnp.int32(0))

    def process_hits(col0, width, buf):
        def filt(h, scnt):
            pos = h * L + lanes
            rv = whits_r[pl.ds(h * L, L)]
            kv = whits_k[pl.ds(h * L, L)]
            m = (pos < cnt) & (rv >= col0) & (rv < col0 + width)
            return _append(scnt, shits_r, shits_k, rv, kv, m)

        scnt = lax.fori_loop(0, (cnt + L - 1) // L, filt, jnp.int32(0))
        nq = (scnt + L - 1) // L

        def drain_group(_, __):
            pltpu.make_async_copy(
                g_hbm.at[pl.ds(0, L * EMBED_DIM)],
                r2_v.at[pl.ds(0, L * EMBED_DIM)], sem).wait()
            return 0

        def extract(q, _):
            pos = q * L + lanes
            vm = pos < scnt
            rv = shits_r[pl.ds(q * L, L)]
            kv = shits_k[pl.ds(q * L, L)]
            rl = jnp.where(vm, rv - col0, 0)
            rowq = (q & (RING - 1)) * L
            for d in range(EMBED_DIM):
                c = (d + lanes) & (EMBED_DIM - 1)
                vals = plsc.load_gather(buf, [c, rl], mask=vm)
                plsc.store_scatter(r2_v, [(rowq + lanes) * EMBED_DIM + c],
                                   vals, mask=vm)
            ksafe = jnp.where(vm, kv * EMBED_DIM, dummy)
            for j in range(L):
                koff = pl.multiple_of(ksafe[j], EMBED_DIM)
                pltpu.async_copy(
                    r2_v.at[pl.ds((rowq + j) * EMBED_DIM, EMBED_DIM)],
                    g_hbm.at[pl.ds(koff, EMBED_DIM)], sem)

            @pl.when((q & (RING - 1)) == RING - 1)
            def _():
                lax.fori_loop(0, RING, drain_group, 0)

            return 0

        lax.fori_loop(0, nq, extract, 0)
        lax.fori_loop(0, nq & (RING - 1), drain_group, 0)

    def main_slab(s, _):
        col0 = lo + s * SLAB
        pltpu.sync_copy(emb_t_hbm.at[:, pl.ds(col0, SLAB)], slab_v)
        process_hits(col0, SLAB, slab_v)
        return 0

    lax.fori_loop(0, SLABS_PER_W, main_slab, 0)

    @pl.when(wid == NW - 1)
    def _():
        pltpu.sync_copy(emb_t_hbm.at[:, pl.ds(EXTRA_SLAB0, SLAB)], slab_v)
        process_hits(jnp.int32(EXTRA_SLAB0), SLAB, slab_v)
        pltpu.sync_copy(emb_t_hbm.at[:, pl.ds(TAIL0, TAIL_W)], tail_v)
        process_hits(jnp.int32(TAIL0), TAIL_W, tail_v)


def _norm_body(outs_t_hbm, g_hbm, out_t_hbm, obuf_t, gbuf, catbuf_t):
    wid = lax.axis_index("s") * NC + lax.axis_index("c")
    lanes = lax.iota(jnp.int32, L)

    for ch in range(ROWS_PER_W // CHUNK):
        base = wid * ROWS_PER_W + ch * CHUNK
        pltpu.sync_copy(outs_t_hbm.at[:, pl.ds(base, CHUNK)], obuf_t)
        pltpu.sync_copy(g_hbm.at[pl.ds(base * EMBED_DIM, CHUNK * EMBED_DIM)],
                        gbuf)

        def group(g, _):
            sl = pl.ds(g * L, L)
            accs = [None] * 8
            for col in range(OUT_DIM):
                v = obuf_t[col, sl]
                a = col & 7
                accs[a] = v * v if accs[a] is None else accs[a] + v * v
            acc = ((accs[0] + accs[1]) + (accs[2] + accs[3])) + (
                (accs[4] + accs[5]) + (accs[6] + accs[7]))
            io = _inv_norm(acc)
            rowbase = (g * L + lanes) * EMBED_DIM
            acc2s = [None] * 4
            for d in range(EMBED_DIM):
                c = (d + lanes) & (EMBED_DIM - 1)
                vals = plsc.load_gather(gbuf, [rowbase + c])
                a = d & 3
                acc2s[a] = (vals * vals if acc2s[a] is None
                            else acc2s[a] + vals * vals)
            acc2 = (acc2s[0] + acc2s[1]) + (acc2s[2] + acc2s[3])
            ie = _inv_norm(acc2)
            for col in range(OUT_DIM):
                catbuf_t[col, sl] = obuf_t[col, sl] * io
            for d in range(EMBED_DIM):
                c = (d + lanes) & (EMBED_DIM - 1)
                vals = plsc.load_gather(gbuf, [rowbase + c])
                plsc.store_scatter(catbuf_t, [OUT_DIM + c, g * L + lanes],
                                   vals * ie)
            return 0

        lax.fori_loop(0, GROUPS, group, 0)
        pltpu.sync_copy(catbuf_t, out_t_hbm.at[:, pl.ds(base, CHUNK)])


@jax.jit
def _crowd_concat(outputs, annotators, embedding):
    emb_t = embedding.T
    outs_t = outputs.T
    mesh = plsc.VectorSubcoreMesh(core_axis_name="c", subcore_axis_name="s")
    params = pltpu.CompilerParams(
        needs_layout_passes=False, use_tc_tiling_on_sc=True)

    g1 = pl.kernel(
        _scan_body,
        out_type=jax.ShapeDtypeStruct((G_ROWS * EMBED_DIM,), jnp.float32),
        mesh=mesh,
        scratch_types=[
            pltpu.VMEM((2048,), jnp.int32),
            pltpu.VMEM((BATCH,), jnp.int32),
            pltpu.VMEM((BATCH,), jnp.int32),
            pltpu.VMEM((BATCH,), jnp.int32),
            pltpu.VMEM((BATCH,), jnp.int32),
            pltpu.VMEM((EMBED_DIM, SLAB), jnp.float32),
            pltpu.VMEM((EMBED_DIM, TAIL_W), jnp.float32),
            pltpu.VMEM((RING * L * EMBED_DIM,), jnp.float32),
            pltpu.SemaphoreType.DMA,
        ],
        compiler_params=params,
    )(annotators, emb_t)

    out_t = pl.kernel(
        _norm_body,
        out_type=jax.ShapeDtypeStruct((CAT_DIM, BATCH), jnp.float32),
        mesh=mesh,
        scratch_types=[
            pltpu.VMEM((OUT_DIM, CHUNK), jnp.float32),
            pltpu.VMEM((CHUNK * EMBED_DIM,), jnp.float32),
            pltpu.VMEM((CAT_DIM, CHUNK), jnp.float32),
        ],
        compiler_params=params,
    )(outs_t, g1)

    return out_t.T


def kernel(outputs, annotators, embedding):
    return _crowd_concat(outputs, annotators, embedding)


# speedup vs baseline: 1.3554x; 1.0860x over previous
import jax
import jax.numpy as jnp
from jax import lax
from jax.experimental import pallas as pl
from jax.experimental.pallas import tpu as pltpu
from jax.experimental.pallas import tpu_sc as plsc

BATCH = 16384
OUT_DIM = 128
EMBED_DIM = 64
CAT_DIM = OUT_DIM + EMBED_DIM
N_ROWS = 1000001
NC, NS, L = 2, 16, 16
NW = NC * NS
ROWS_PER_W = BATCH // NW
CHUNK = 256
GROUPS = CHUNK // L

SLAB = 512
SLABS_PER_W = 61
W_RANGE = SLABS_PER_W * SLAB
EXTRA_SLAB0 = NW * SLABS_PER_W * SLAB
TAIL0 = EXTRA_SLAB0 + SLAB
TAIL_W = N_ROWS - TAIL0
G_ROWS = BATCH + NW
RING = 16

_RSQRT_MAGIC = 0x5F3759DF


def _inv_norm(s):
    s_safe = jnp.maximum(s, jnp.float32(1.2e-38))
    y = plsc.bitcast(
        jnp.int32(_RSQRT_MAGIC) - (plsc.bitcast(s_safe, jnp.int32) >> 1),
        jnp.float32)
    for _ in range(3):
        y = y * (jnp.float32(1.5) - jnp.float32(0.5) * s_safe * y * y)
    n = s * y
    return jnp.float32(1.0) / jnp.maximum(n, jnp.float32(1e-12))


def _scan_body(ann_hbm, emb_t_hbm, g_hbm,
               abuf, whits_r, whits_k, shits_r, shits_k,
               slab_v, tail_v, r2_v, sem, sem_s):
    wid = lax.axis_index("s") * NC + lax.axis_index("c")
    lo = wid * W_RANGE
    hi = jnp.where(wid == NW - 1, jnp.int32(N_ROWS), lo + W_RANGE)
    lanes = lax.iota(jnp.int32, L)
    dummy = (BATCH + wid) * EMBED_DIM

    def _append(cnt, dst_r, dst_k, rv, kv, m):
        plsc.store_compressed(dst_r.at[pl.ds(cnt, L)], rv, mask=m)
        plsc.store_compressed(dst_k.at[pl.ds(cnt, L)], kv, mask=m)
        npop = plsc.all_reduce_population_count(m)
        return cnt + npop[0]

    def scan_chunk(c8, cnt):
        pltpu.sync_copy(ann_hbm.at[pl.ds(c8 * 2048, 2048)], abuf)

        def scan_group(g, cnt):
            rv = abuf[pl.ds(g * L, L)]
            kv = c8 * 2048 + g * L + lanes
            m = (rv >= lo) & (rv < hi)
            return _append(cnt, whits_r, whits_k, rv, kv, m)

        return lax.fori_loop(0, 2048 // L, scan_group, cnt)

    cnt = lax.fori_loop(0, BATCH // 2048, scan_chunk, jnp.int32(0))

    def process_hits(col0, width, buf, dma=None):
        def filt(h, scnt):
            pos = h * L + lanes
            rv = whits_r[pl.ds(h * L, L)]
            kv = whits_k[pl.ds(h * L, L)]
            m = (pos < cnt) & (rv >= col0) & (rv < col0 + width)
            return _append(scnt, shits_r, shits_k, rv, kv, m)

        scnt = lax.fori_loop(0, (cnt + L - 1) // L, filt, jnp.int32(0))
        nq = (scnt + L - 1) // L
        if dma is not None:
            dma.wait()

        def drain_group(_, __):
            pltpu.make_async_copy(
                g_hbm.at[pl.ds(0, L * EMBED_DIM)],
                r2_v.at[pl.ds(0, L * EMBED_DIM)], sem).wait()
            return 0

        def extract(q, _):
            pos = q * L + lanes
            vm = pos < scnt
            rv = shits_r[pl.ds(q * L, L)]
            kv = shits_k[pl.ds(q * L, L)]
            rl = jnp.where(vm, rv - col0, 0)
            rowq = (q & (RING - 1)) * L
            for d in range(EMBED_DIM):
                c = (d + lanes) & (EMBED_DIM - 1)
                vals = plsc.load_gather(buf, [c, rl], mask=vm)
                plsc.store_scatter(r2_v, [(rowq + lanes) * EMBED_DIM + c],
                                   vals, mask=vm)
            ksafe = jnp.where(vm, kv * EMBED_DIM, dummy)
            for j in range(L):
                koff = pl.multiple_of(ksafe[j], EMBED_DIM)
                pltpu.async_copy(
                    r2_v.at[pl.ds((rowq + j) * EMBED_DIM, EMBED_DIM)],
                    g_hbm.at[pl.ds(koff, EMBED_DIM)], sem)

            @pl.when((q & (RING - 1)) == RING - 1)
            def _():
                lax.fori_loop(0, RING, drain_group, 0)

            return 0

        lax.fori_loop(0, nq, extract, 0)
        lax.fori_loop(0, nq & (RING - 1), drain_group, 0)

    def main_slab(s, _):
        col0 = lo + s * SLAB
        dma = pltpu.async_copy(emb_t_hbm.at[:, pl.ds(col0, SLAB)],
                               slab_v, sem_s)
        process_hits(col0, SLAB, slab_v, dma)
        return 0

    lax.fori_loop(0, SLABS_PER_W, main_slab, 0)

    @pl.when(wid == NW - 1)
    def _():
        pltpu.sync_copy(emb_t_hbm.at[:, pl.ds(EXTRA_SLAB0, SLAB)], slab_v)
        process_hits(jnp.int32(EXTRA_SLAB0), SLAB, slab_v)
        pltpu.sync_copy(emb_t_hbm.at[:, pl.ds(TAIL0, TAIL_W)], tail_v)
        process_hits(jnp.int32(TAIL0), TAIL_W, tail_v)


def _norm_body(outs_t_hbm, g_hbm, out_t_hbm, obuf_t, gbuf, catbuf_t):
    wid = lax.axis_index("s") * NC + lax.axis_index("c")
    lanes = lax.iota(jnp.int32, L)

    for ch in range(ROWS_PER_W // CHUNK):
        base = wid * ROWS_PER_W + ch * CHUNK
        pltpu.sync_copy(outs_t_hbm.at[:, pl.ds(base, CHUNK)], obuf_t)
        pltpu.sync_copy(g_hbm.at[pl.ds(base * EMBED_DIM, CHUNK * EMBED_DIM)],
                        gbuf)

        def group(g, _):
            sl = pl.ds(g * L, L)
            accs = [None] * 8
            for col in range(OUT_DIM):
                v = obuf_t[col, sl]
                a = col & 7
                accs[a] = v * v if accs[a] is None else accs[a] + v * v
            acc = ((accs[0] + accs[1]) + (accs[2] + accs[3])) + (
                (accs[4] + accs[5]) + (accs[6] + accs[7]))
            io = _inv_norm(acc)
            rowbase = (g * L + lanes) * EMBED_DIM
            acc2s = [None] * 4
            for d in range(EMBED_DIM):
                c = (d + lanes) & (EMBED_DIM - 1)
                vals = plsc.load_gather(gbuf, [rowbase + c])
                a = d & 3
                acc2s[a] = (vals * vals if acc2s[a] is None
                            else acc2s[a] + vals * vals)
            acc2 = (acc2s[0] + acc2s[1]) + (acc2s[2] + acc2s[3])
            ie = _inv_norm(acc2)
            for col in range(OUT_DIM):
                catbuf_t[col, sl] = obuf_t[col, sl] * io
            for d in range(EMBED_DIM):
                c = (d + lanes) & (EMBED_DIM - 1)
                vals = plsc.load_gather(gbuf, [rowbase + c])
                plsc.store_scatter(catbuf_t, [OUT_DIM + c, g * L + lanes],
                                   vals * ie)
            return 0

        lax.fori_loop(0, GROUPS, group, 0)
        pltpu.sync_copy(catbuf_t, out_t_hbm.at[:, pl.ds(base, CHUNK)])


@jax.jit
def _crowd_concat(outputs, annotators, embedding):
    emb_t = embedding.T
    outs_t = outputs.T
    mesh = plsc.VectorSubcoreMesh(core_axis_name="c", subcore_axis_name="s")
    params = pltpu.CompilerParams(
        needs_layout_passes=False, use_tc_tiling_on_sc=True)

    g1 = pl.kernel(
        _scan_body,
        out_type=jax.ShapeDtypeStruct((G_ROWS * EMBED_DIM,), jnp.float32),
        mesh=mesh,
        scratch_types=[
            pltpu.VMEM((2048,), jnp.int32),
            pltpu.VMEM((BATCH,), jnp.int32),
            pltpu.VMEM((BATCH,), jnp.int32),
            pltpu.VMEM((BATCH,), jnp.int32),
            pltpu.VMEM((BATCH,), jnp.int32),
            pltpu.VMEM((EMBED_DIM, SLAB), jnp.float32),
            pltpu.VMEM((EMBED_DIM, TAIL_W), jnp.float32),
            pltpu.VMEM((RING * L * EMBED_DIM,), jnp.float32),
            pltpu.SemaphoreType.DMA,
            pltpu.SemaphoreType.DMA,
        ],
        compiler_params=params,
    )(annotators, emb_t)

    out_t = pl.kernel(
        _norm_body,
        out_type=jax.ShapeDtypeStruct((CAT_DIM, BATCH), jnp.float32),
        mesh=mesh,
        scratch_types=[
            pltpu.VMEM((OUT_DIM, CHUNK), jnp.float32),
            pltpu.VMEM((CHUNK * EMBED_DIM,), jnp.float32),
            pltpu.VMEM((CAT_DIM, CHUNK), jnp.float32),
        ],
        compiler_params=params,
    )(outs_t, g1)

    return out_t.T


def kernel(outputs, annotators, embedding):
    return _crowd_concat(outputs, annotators, embedding)


# speedup vs baseline: 1.4357x; 1.0592x over previous
import jax
import jax.numpy as jnp
from jax import lax
from jax.experimental import pallas as pl
from jax.experimental.pallas import tpu as pltpu
from jax.experimental.pallas import tpu_sc as plsc

BATCH = 16384
OUT_DIM = 128
EMBED_DIM = 64
CAT_DIM = OUT_DIM + EMBED_DIM
N_ROWS = 1000001
NC, NS, L = 2, 16, 16
NW = NC * NS
ROWS_PER_W = BATCH // NW
CHUNK = 128
GROUPS = CHUNK // L

SLAB = 512
SLABS_PER_W = 61
W_RANGE = SLABS_PER_W * SLAB
EXTRA_SLAB0 = NW * SLABS_PER_W * SLAB
TAIL0 = EXTRA_SLAB0 + SLAB
TAIL_W = N_ROWS - TAIL0
G_ROWS = BATCH + NW
RING = 16

_RSQRT_MAGIC = 0x5F3759DF


def _inv_norm(s):
    s_safe = jnp.maximum(s, jnp.float32(1.2e-38))
    y = plsc.bitcast(
        jnp.int32(_RSQRT_MAGIC) - (plsc.bitcast(s_safe, jnp.int32) >> 1),
        jnp.float32)
    for _ in range(3):
        y = y * (jnp.float32(1.5) - jnp.float32(0.5) * s_safe * y * y)
    n = s * y
    return jnp.float32(1.0) / jnp.maximum(n, jnp.float32(1e-12))


def _scan_body(ann_hbm, emb_t_hbm, g_hbm,
               abuf, whits_r, whits_k, shits_r, shits_k,
               slab_v, tail_v, r2_v, sem, sem_s):
    wid = lax.axis_index("s") * NC + lax.axis_index("c")
    lo = wid * W_RANGE
    hi = jnp.where(wid == NW - 1, jnp.int32(N_ROWS), lo + W_RANGE)
    lanes = lax.iota(jnp.int32, L)
    dummy = (BATCH + wid) * EMBED_DIM

    def _append(cnt, dst_r, dst_k, rv, kv, m):
        plsc.store_compressed(dst_r.at[pl.ds(cnt, L)], rv, mask=m)
        plsc.store_compressed(dst_k.at[pl.ds(cnt, L)], kv, mask=m)
        npop = plsc.all_reduce_population_count(m)
        return cnt + npop[0]

    def scan_chunk(c8, cnt):
        pltpu.sync_copy(ann_hbm.at[pl.ds(c8 * 2048, 2048)], abuf)

        def scan_group(g, cnt):
            rv = abuf[pl.ds(g * L, L)]
            kv = c8 * 2048 + g * L + lanes
            m = (rv >= lo) & (rv < hi)
            return _append(cnt, whits_r, whits_k, rv, kv, m)

        return lax.fori_loop(0, 2048 // L, scan_group, cnt)

    cnt = lax.fori_loop(0, BATCH // 2048, scan_chunk, jnp.int32(0))

    def process_hits(col0, width, buf, dma=None):
        def filt(h, scnt):
            pos = h * L + lanes
            rv = whits_r[pl.ds(h * L, L)]
            kv = whits_k[pl.ds(h * L, L)]
            m = (pos < cnt) & (rv >= col0) & (rv < col0 + width)
            return _append(scnt, shits_r, shits_k, rv, kv, m)

        scnt = lax.fori_loop(0, (cnt + L - 1) // L, filt, jnp.int32(0))
        nq = (scnt + L - 1) // L
        if dma is not None:
            dma.wait()

        def drain_group(_, __):
            pltpu.make_async_copy(
                g_hbm.at[pl.ds(0, L * EMBED_DIM)],
                r2_v.at[pl.ds(0, L * EMBED_DIM)], sem).wait()
            return 0

        def extract(q, _):
            pos = q * L + lanes
            vm = pos < scnt
            rv = shits_r[pl.ds(q * L, L)]
            kv = shits_k[pl.ds(q * L, L)]
            rl = jnp.where(vm, rv - col0, 0)
            rowq = (q & (RING - 1)) * L
            for d in range(EMBED_DIM):
                c = (d + lanes) & (EMBED_DIM - 1)
                vals = plsc.load_gather(buf, [c, rl], mask=vm)
                plsc.store_scatter(r2_v, [(rowq + lanes) * EMBED_DIM + c],
                                   vals, mask=vm)
            ksafe = jnp.where(vm, kv * EMBED_DIM, dummy)
            for j in range(L):
                koff = pl.multiple_of(ksafe[j], EMBED_DIM)
                pltpu.async_copy(
                    r2_v.at[pl.ds((rowq + j) * EMBED_DIM, EMBED_DIM)],
                    g_hbm.at[pl.ds(koff, EMBED_DIM)], sem)

            @pl.when((q & (RING - 1)) == RING - 1)
            def _():
                lax.fori_loop(0, RING, drain_group, 0)

            return 0

        lax.fori_loop(0, nq, extract, 0)
        lax.fori_loop(0, nq & (RING - 1), drain_group, 0)

    def main_slab(s, _):
        col0 = lo + s * SLAB
        dma = pltpu.async_copy(emb_t_hbm.at[:, pl.ds(col0, SLAB)],
                               slab_v, sem_s)
        process_hits(col0, SLAB, slab_v, dma)
        return 0

    lax.fori_loop(0, SLABS_PER_W, main_slab, 0)

    @pl.when(wid == NW - 1)
    def _():
        pltpu.sync_copy(emb_t_hbm.at[:, pl.ds(EXTRA_SLAB0, SLAB)], slab_v)
        process_hits(jnp.int32(EXTRA_SLAB0), SLAB, slab_v)
        pltpu.sync_copy(emb_t_hbm.at[:, pl.ds(TAIL0, TAIL_W)], tail_v)
        process_hits(jnp.int32(TAIL0), TAIL_W, tail_v)


def _norm_body(outs_t_hbm, g_hbm, out_t_hbm,
               obuf_a, obuf_b, gbuf_a, gbuf_b, cat_a, cat_b,
               semi, semo_a, semo_b):
    wid = lax.axis_index("s") * NC + lax.axis_index("c")
    lanes = lax.iota(jnp.int32, L)
    nch = ROWS_PER_W // CHUNK
    bufs = [(obuf_a, gbuf_a, cat_a, semo_a), (obuf_b, gbuf_b, cat_b, semo_b)]

    def start_in(ch):
        base = wid * ROWS_PER_W + ch * CHUNK
        obuf_t, gbuf, _, _ = bufs[ch % 2]
        pltpu.async_copy(outs_t_hbm.at[:, pl.ds(base, CHUNK)], obuf_t, semi)
        pltpu.async_copy(
            g_hbm.at[pl.ds(base * EMBED_DIM, CHUNK * EMBED_DIM)], gbuf, semi)

    start_in(0)
    for ch in range(nch):
        base = wid * ROWS_PER_W + ch * CHUNK
        obuf_t, gbuf, catbuf_t, semo = bufs[ch % 2]
        pltpu.make_async_copy(
            outs_t_hbm.at[:, pl.ds(0, CHUNK)], obuf_t, semi).wait()
        pltpu.make_async_copy(
            g_hbm.at[pl.ds(0, CHUNK * EMBED_DIM)], gbuf, semi).wait()
        if ch + 1 < nch:
            start_in(ch + 1)
        if ch >= 2:
            pltpu.make_async_copy(
                out_t_hbm.at[:, pl.ds(0, CHUNK)], catbuf_t, semo).wait()

        def group(g, _):
            sl = pl.ds(g * L, L)
            accs = [None] * 8
            for col in range(OUT_DIM):
                v = obuf_t[col, sl]
                a = col & 7
                accs[a] = v * v if accs[a] is None else accs[a] + v * v
            acc = ((accs[0] + accs[1]) + (accs[2] + accs[3])) + (
                (accs[4] + accs[5]) + (accs[6] + accs[7]))
            io = _inv_norm(acc)
            rowbase = (g * L + lanes) * EMBED_DIM
            acc2s = [None] * 4
            for d in range(EMBED_DIM):
                c = (d + lanes) & (EMBED_DIM - 1)
                vals = plsc.load_gather(gbuf, [rowbase + c])
                a = d & 3
                acc2s[a] = (vals * vals if acc2s[a] is None
                            else acc2s[a] + vals * vals)
            acc2 = (acc2s[0] + acc2s[1]) + (acc2s[2] + acc2s[3])
            ie = _inv_norm(acc2)
            for col in range(OUT_DIM):
                catbuf_t[col, sl] = obuf_t[col, sl] * io
            for d in range(EMBED_DIM):
                c = (d + lanes) & (EMBED_DIM - 1)
                vals = plsc.load_gather(gbuf, [rowbase + c])
                plsc.store_scatter(catbuf_t, [OUT_DIM + c, g * L + lanes],
                                   vals * ie)
            return 0

        lax.fori_loop(0, GROUPS, group, 0)
        pltpu.async_copy(catbuf_t, out_t_hbm.at[:, pl.ds(base, CHUNK)], semo)

    for ch in (nch - 2, nch - 1):
        catbuf_t, semo = bufs[ch % 2][2], bufs[ch % 2][3]
        pltpu.make_async_copy(
            out_t_hbm.at[:, pl.ds(0, CHUNK)], catbuf_t, semo).wait()


@jax.jit
def _crowd_concat(outputs, annotators, embedding):
    emb_t = embedding.T
    outs_t = outputs.T
    mesh = plsc.VectorSubcoreMesh(core_axis_name="c", subcore_axis_name="s")
    params = pltpu.CompilerParams(
        needs_layout_passes=False, use_tc_tiling_on_sc=True)

    g1 = pl.kernel(
        _scan_body,
        out_type=jax.ShapeDtypeStruct((G_ROWS * EMBED_DIM,), jnp.float32),
        mesh=mesh,
        scratch_types=[
            pltpu.VMEM((2048,), jnp.int32),
            pltpu.VMEM((BATCH,), jnp.int32),
            pltpu.VMEM((BATCH,), jnp.int32),
            pltpu.VMEM((BATCH,), jnp.int32),
            pltpu.VMEM((BATCH,), jnp.int32),
            pltpu.VMEM((EMBED_DIM, SLAB), jnp.float32),
            pltpu.VMEM((EMBED_DIM, TAIL_W), jnp.float32),
            pltpu.VMEM((RING * L * EMBED_DIM,), jnp.float32),
            pltpu.SemaphoreType.DMA,
            pltpu.SemaphoreType.DMA,
        ],
        compiler_params=params,
    )(annotators, emb_t)

    out_t = pl.kernel(
        _norm_body,
        out_type=jax.ShapeDtypeStruct((CAT_DIM, BATCH), jnp.float32),
        mesh=mesh,
        scratch_types=[
            pltpu.VMEM((OUT_DIM, CHUNK), jnp.float32),
            pltpu.VMEM((OUT_DIM, CHUNK), jnp.float32),
            pltpu.VMEM((CHUNK * EMBED_DIM,), jnp.float32),
            pltpu.VMEM((CHUNK * EMBED_DIM,), jnp.float32),
            pltpu.VMEM((CAT_DIM, CHUNK), jnp.float32),
            pltpu.VMEM((CAT_DIM, CHUNK), jnp.float32),
            pltpu.SemaphoreType.DMA,
            pltpu.SemaphoreType.DMA,
            pltpu.SemaphoreType.DMA,
        ],
        compiler_params=params,
    )(outs_t, g1)

    return out_t.T


def kernel(outputs, annotators, embedding):
    return _crowd_concat(outputs, annotators, embedding)
